# speedup vs baseline: 1.2709x; 1.2709x over previous
import functools

import jax
import jax.numpy as jnp
from jax.experimental import pallas as pl
from jax.experimental.pallas import tpu as pltpu

HIDDEN = 1024
NUM_EXPERTS = 8
TOP_K = 2
MOE_FF = 512
BT = 256


def _moe_dense_kernel(x_ref, gw_ref, wg_ref, wu_ref, wd_ref, out_ref):
    e = pl.program_id(1)
    xb = x_ref[...]

    logits = jnp.dot(xb, gw_ref[...].T, preferred_element_type=jnp.float32)
    m = jnp.max(logits, axis=1, keepdims=True)
    p = jnp.exp(logits - m)
    prob = p / jnp.sum(p, axis=1, keepdims=True)
    iota_e = jax.lax.broadcasted_iota(jnp.int32, prob.shape, 1)
    i1 = jnp.argmax(prob, axis=1).astype(jnp.int32)
    w1 = jnp.max(prob, axis=1)
    masked = jnp.where(iota_e == i1[:, None], -1.0, prob)
    i2 = jnp.argmax(masked, axis=1).astype(jnp.int32)
    w2 = jnp.max(masked, axis=1)
    we = (jnp.where(i1 == e, w1, 0.0) + jnp.where(i2 == e, w2, 0.0)) / (w1 + w2)

    g = jnp.dot(xb, wg_ref[0].T, preferred_element_type=jnp.float32)
    u = jnp.dot(xb, wu_ref[0].T, preferred_element_type=jnp.float32)
    act = g * jax.nn.sigmoid(g) * u
    o = jnp.dot(act, wd_ref[0].T, preferred_element_type=jnp.float32)
    contrib = o * we[:, None]

    @pl.when(e == 0)
    def _():
        out_ref[...] = contrib

    @pl.when(e > 0)
    def _():
        out_ref[...] += contrib


@functools.partial(jax.jit, static_argnames=("interpret",))
def _moe(x, gate_w, Wg, Wu, Wd, interpret=False):
    T, H = x.shape
    E, F, _ = Wg.shape
    grid = (T // BT, E)
    return pl.pallas_call(
        _moe_dense_kernel,
        grid=grid,
        in_specs=[
            pl.BlockSpec((BT, H), lambda t, e: (t, 0)),
            pl.BlockSpec((E, H), lambda t, e: (0, 0)),
            pl.BlockSpec((1, F, H), lambda t, e: (e, 0, 0)),
            pl.BlockSpec((1, F, H), lambda t, e: (e, 0, 0)),
            pl.BlockSpec((1, H, F), lambda t, e: (e, 0, 0)),
        ],
        out_specs=pl.BlockSpec((BT, H), lambda t, e: (t, 0)),
        out_shape=jax.ShapeDtypeStruct((T, H), jnp.float32),
        interpret=interpret,
    )(x, gate_w, Wg, Wu, Wd)


def kernel(hidden_states, gate_w, Wg, Wu, Wd):
    b, s, h = hidden_states.shape
    x = hidden_states.reshape(-1, h)
    out = _moe(x, gate_w, Wg, Wu, Wd)
    return out.reshape(b, s, h)
